# baseline (speedup 1.0000x reference)
import functools

import jax
import jax.numpy as jnp
from jax import lax
from jax.experimental import pallas as pl
from jax.experimental.pallas import tpu as pltpu
from jax.experimental.pallas import tpu_sc as plsc

VOCAB = 1000000
EMBED_DIM = 64
BATCH = 4096
HIST = 200
DROPOUT = 0.1

N = BATCH * HIST
NC, NS, LANES = 2, 16, 16
NW = NC * NS
N_PER_W = N // NW
CHUNK = 512
GROUP = 128
NGROUP = CHUNK // GROUP
NSTEP = N_PER_W // CHUNK


def _sc_body(words_hbm, weight_hbm, mvals_hbm, out_hbm,
             idx_v, rows_v, mval_v, sem):
    wid = lax.axis_index("s") * NC + lax.axis_index("c")

    def step(s, carry):
        base = pl.multiple_of(wid * N_PER_W + s * CHUNK, CHUNK)
        row0 = pl.multiple_of(base // GROUP, NGROUP)
        pltpu.sync_copy(words_hbm.at[pl.ds(row0, NGROUP)], idx_v)
        descs = []
        for g in range(NGROUP):
            descs.append(pltpu.async_copy(
                weight_hbm.at[idx_v.at[g]],
                rows_v.at[pl.ds(g * GROUP, GROUP)], sem))
            descs.append(pltpu.async_copy(
                mvals_hbm.at[idx_v.at[g]],
                mval_v.at[pl.ds(g * GROUP, GROUP)], sem))
        for d in descs:
            d.wait()

        def scale_row(r, carry):
            mv = mval_v[pl.ds(r, LANES)]
            m = jnp.full((LANES,), mv[0])
            for j in range(EMBED_DIM // LANES):
                sl = pl.ds(j * LANES, LANES)
                rows_v[r, sl] = rows_v[r, sl] * m
            return carry

        lax.fori_loop(0, CHUNK, scale_row, None)
        pltpu.sync_copy(rows_v, out_hbm.at[pl.ds(base, CHUNK)])
        return carry

    lax.fori_loop(0, NSTEP, step, None)


@jax.jit
def kernel(words, weight):
    mask_key = jax.random.key(42)
    keep = jax.random.bernoulli(mask_key, 1.0 - DROPOUT, (VOCAB, 1))
    mvals = (keep.astype(weight.dtype) / (1.0 - DROPOUT)).reshape(VOCAB)

    words2d = words.reshape(N // GROUP, GROUP)
    mesh = plsc.VectorSubcoreMesh(core_axis_name="c", subcore_axis_name="s")
    out = pl.kernel(
        _sc_body,
        out_type=jax.ShapeDtypeStruct((N, EMBED_DIM), jnp.float32),
        mesh=mesh,
        scratch_types=[
            pltpu.VMEM((NGROUP, GROUP), jnp.int32),
            pltpu.VMEM((CHUNK, EMBED_DIM), jnp.float32),
            pltpu.VMEM((CHUNK + LANES,), jnp.float32),
            pltpu.SemaphoreType.DMA,
        ],
        compiler_params=pltpu.CompilerParams(use_tc_tiling_on_sc=False),
    )(words2d, weight, mvals)
    return out.reshape(BATCH, HIST, EMBED_DIM)


# speedup vs baseline: 1.0542x; 1.0542x over previous
import functools

import jax
import jax.numpy as jnp
from jax import lax
from jax.experimental import pallas as pl
from jax.experimental.pallas import tpu as pltpu
from jax.experimental.pallas import tpu_sc as plsc

VOCAB = 1000000
EMBED_DIM = 64
BATCH = 4096
HIST = 200
DROPOUT = 0.1

N = BATCH * HIST
NC, NS, LANES = 2, 16, 16
NW = NC * NS
N_PER_W = N // NW
CHUNK = 512
GROUP = 128
NGROUP = CHUNK // GROUP
NSTEP = N_PER_W // CHUNK


def _sc_body(words_hbm, weight_hbm, mvals_hbm, out_hbm,
             idx_v, rows_v, mval_v, sem):
    wid = lax.axis_index("s") * NC + lax.axis_index("c")

    def step(s, carry):
        base = pl.multiple_of(wid * N_PER_W + s * CHUNK, CHUNK)
        row0 = pl.multiple_of(base // GROUP, NGROUP)
        pltpu.sync_copy(words_hbm.at[pl.ds(row0, NGROUP)], idx_v)
        descs = []
        for g in range(NGROUP):
            descs.append(pltpu.async_copy(
                weight_hbm.at[idx_v.at[g]],
                rows_v.at[pl.ds(g * GROUP, GROUP)], sem))
            descs.append(pltpu.async_copy(
                mvals_hbm.at[idx_v.at[g]],
                mval_v.at[pl.ds(g * GROUP, GROUP)], sem))
        for d in descs:
            d.wait()

        def scale_grp(g, carry):
            r0 = g * LANES
            mv = mval_v[pl.ds(r0, LANES)]
            for l in range(LANES):
                m = jnp.full((LANES,), mv[l])
                for j in range(EMBED_DIM // LANES):
                    sl = pl.ds(j * LANES, LANES)
                    rows_v[r0 + l, sl] = rows_v[r0 + l, sl] * m
            return carry

        lax.fori_loop(0, CHUNK // LANES, scale_grp, None)
        pltpu.sync_copy(rows_v, out_hbm.at[pl.ds(base, CHUNK)])
        return carry

    lax.fori_loop(0, NSTEP, step, None)


@jax.jit
def kernel(words, weight):
    mask_key = jax.random.key(42)
    keep = jax.random.bernoulli(mask_key, 1.0 - DROPOUT, (VOCAB, 1))
    mvals = (keep.astype(weight.dtype) / (1.0 - DROPOUT)).reshape(VOCAB)

    words2d = words.reshape(N // GROUP, GROUP)
    mesh = plsc.VectorSubcoreMesh(core_axis_name="c", subcore_axis_name="s")
    out = pl.kernel(
        _sc_body,
        out_type=jax.ShapeDtypeStruct((N, EMBED_DIM), jnp.float32),
        mesh=mesh,
        scratch_types=[
            pltpu.VMEM((NGROUP, GROUP), jnp.int32),
            pltpu.VMEM((CHUNK, EMBED_DIM), jnp.float32),
            pltpu.VMEM((CHUNK + LANES,), jnp.float32),
            pltpu.SemaphoreType.DMA,
        ],
        compiler_params=pltpu.CompilerParams(use_tc_tiling_on_sc=False),
    )(words2d, weight, mvals)
    return out.reshape(BATCH, HIST, EMBED_DIM)


# speedup vs baseline: 1.1008x; 1.0442x over previous
import functools

import jax
import jax.numpy as jnp
from jax import lax
from jax.experimental import pallas as pl
from jax.experimental.pallas import tpu as pltpu
from jax.experimental.pallas import tpu_sc as plsc

VOCAB = 1000000
EMBED_DIM = 64
BATCH = 4096
HIST = 200
DROPOUT = 0.1

N = BATCH * HIST
NC, NS, LANES = 2, 16, 16
NW = NC * NS
N_PER_W = N // NW
CHUNK = 256
GROUP = 128
NGROUP = CHUNK // GROUP
NSTEP = N_PER_W // CHUNK
NSLOT = 4
DEPTH = 2


def _sc_body(words_hbm, weight_hbm, mvals_hbm, out_hbm,
             idx_v, rows_v, mval_v, gsems, wsems):
    wid = lax.axis_index("s") * NC + lax.axis_index("c")
    w0 = wid * N_PER_W

    def stage_and_gather(t, b):
        base = pl.multiple_of(w0 + t * CHUNK, CHUNK)
        row0 = pl.multiple_of(base // GROUP, NGROUP)
        pltpu.sync_copy(words_hbm.at[pl.ds(row0, NGROUP)], idx_v.at[b])
        for g in range(NGROUP):
            pltpu.async_copy(
                weight_hbm.at[idx_v.at[b, g]],
                rows_v.at[b, pl.ds(g * GROUP, GROUP)], gsems[b])
            pltpu.async_copy(
                mvals_hbm.at[idx_v.at[b, g]],
                mval_v.at[b, pl.ds(g * GROUP, GROUP)], gsems[b])

    def wait_gathers(b):
        for g in range(NGROUP):
            pltpu.make_async_copy(
                weight_hbm.at[idx_v.at[b, g]],
                rows_v.at[b, pl.ds(g * GROUP, GROUP)], gsems[b]).wait()
            pltpu.make_async_copy(
                mvals_hbm.at[idx_v.at[b, g]],
                mval_v.at[b, pl.ds(g * GROUP, GROUP)], gsems[b]).wait()

    def wait_writeback(b):
        pltpu.make_async_copy(
            rows_v.at[b], out_hbm.at[pl.ds(0, CHUNK)], wsems[b]).wait()

    for t in range(DEPTH):
        stage_and_gather(t, t)

    def outer(s, carry):
        for b in range(NSLOT):
            t = s + b
            bg = (b + DEPTH) % NSLOT

            @pl.when(t + DEPTH < NSTEP)
            def _():
                @pl.when(t >= NSLOT - DEPTH)
                def _():
                    wait_writeback(bg)
                stage_and_gather(t + DEPTH, bg)

            wait_gathers(b)

            for g16 in range(CHUNK // LANES):
                r0 = g16 * LANES
                mv = mval_v[b, pl.ds(r0, LANES)]
                for l in range(LANES):
                    m = jnp.full((LANES,), mv[l])
                    for j in range(EMBED_DIM // LANES):
                        sl = pl.ds(j * LANES, LANES)
                        rows_v[b, r0 + l, sl] = rows_v[b, r0 + l, sl] * m

            base = pl.multiple_of(w0 + t * CHUNK, CHUNK)
            pltpu.async_copy(rows_v.at[b], out_hbm.at[pl.ds(base, CHUNK)],
                             wsems[b])
        return carry

    lax.fori_loop(0, NSTEP // NSLOT, lambda i, c: outer(i * NSLOT, c), None)

    for b in range(NSLOT):
        wait_writeback(b)


@jax.jit
def kernel(words, weight):
    mask_key = jax.random.key(42)
    keep = jax.random.bernoulli(mask_key, 1.0 - DROPOUT, (VOCAB, 1))
    mvals = (keep.astype(weight.dtype) / (1.0 - DROPOUT)).reshape(VOCAB)

    words2d = words.reshape(N // GROUP, GROUP)
    mesh = plsc.VectorSubcoreMesh(core_axis_name="c", subcore_axis_name="s")

    def body(words_hbm, weight_hbm, mvals_hbm, out_hbm,
             idx_v, rows_v, mval_v,
             g0, g1, g2, g3, ws0, ws1, ws2, ws3):
        _sc_body(words_hbm, weight_hbm, mvals_hbm, out_hbm,
                 idx_v, rows_v, mval_v,
                 (g0, g1, g2, g3), (ws0, ws1, ws2, ws3))

    out = pl.kernel(
        body,
        out_type=jax.ShapeDtypeStruct((N, EMBED_DIM), jnp.float32),
        mesh=mesh,
        scratch_types=[
            pltpu.VMEM((NSLOT, NGROUP, GROUP), jnp.int32),
            pltpu.VMEM((NSLOT, CHUNK, EMBED_DIM), jnp.float32),
            pltpu.VMEM((NSLOT, CHUNK + LANES), jnp.float32),
        ] + [pltpu.SemaphoreType.DMA] * (2 * NSLOT),
        compiler_params=pltpu.CompilerParams(use_tc_tiling_on_sc=False),
    )(words2d, weight, mvals)
    return out.reshape(BATCH, HIST, EMBED_DIM)
